# baseline (speedup 1.0000x reference)
import functools

import jax
import jax.numpy as jnp
from jax import lax
from jax.experimental import pallas as pl
from jax.experimental.pallas import tpu as pltpu
from jax.experimental.pallas import tpu_sc as plsc


def _gate_kernel(x_ref, w_ref, b_ref, gate_ref):
    x = x_ref[...]
    w = w_ref[...]
    gate = jax.lax.dot_general(
        w, x,
        dimension_numbers=(((1,), (1,)), ((), ())),
        preferred_element_type=jnp.float32,
    ) + b_ref[:, :1]
    gate_ref[...] = gate[None]


def _gate_matmul(inp, W, b, nw):
    n_tok, d_model = inp.shape
    n_expert = W.shape[0]
    per_w = n_tok // nw
    return pl.pallas_call(
        _gate_kernel,
        grid=(nw,),
        in_specs=[
            pl.BlockSpec((per_w, d_model), lambda i: (i, 0)),
            pl.BlockSpec((n_expert, d_model), lambda i: (0, 0)),
            pl.BlockSpec((n_expert, 128), lambda i: (0, 0)),
        ],
        out_specs=pl.BlockSpec((1, n_expert, per_w), lambda i: (i, 0, 0)),
        out_shape=jax.ShapeDtypeStruct((nw, n_expert, per_w), jnp.float32),
    )(inp, W, jnp.broadcast_to(b[:, None], (n_expert, 128)))


def _make_sc_topk(n_tok, n_expert, nc, ns):
    nw = nc * ns
    per_w = n_tok // nw
    n_group = per_w // 16
    mesh = plsc.VectorSubcoreMesh(core_axis_name="c", subcore_axis_name="s")

    @functools.partial(
        pl.kernel, mesh=mesh,
        out_type=[
            jax.ShapeDtypeStruct((n_tok,), jnp.int32),
            jax.ShapeDtypeStruct((n_tok,), jnp.int32),
            jax.ShapeDtypeStruct((n_tok,), jnp.float32),
            jax.ShapeDtypeStruct((n_tok,), jnp.float32),
        ],
        scratch_types=[
            pltpu.VMEM((n_expert, per_w), jnp.float32),
            pltpu.VMEM((per_w,), jnp.int32),
            pltpu.VMEM((per_w,), jnp.int32),
            pltpu.VMEM((per_w,), jnp.float32),
            pltpu.VMEM((per_w,), jnp.float32),
        ],
    )
    def sc_topk(gate_hbm, i1_hbm, i2_hbm, s1_hbm, s2_hbm,
                gate_v, i1_v, i2_v, s1_v, s2_v):
        wid = lax.axis_index("s") * nc + lax.axis_index("c")
        base = wid * per_w
        pltpu.sync_copy(gate_hbm.at[wid], gate_v)

        neg_huge = jnp.float32(-3.4e38)
        for g in range(n_group):
            sl = slice(g * 16, g * 16 + 16)
            m1 = gate_v[0, sl]
            i1 = jnp.zeros((16,), jnp.int32)
            m2 = jnp.full((16,), neg_huge, jnp.float32)
            i2 = jnp.zeros((16,), jnp.int32)
            for e in range(1, n_expert):
                v = gate_v[e, sl]
                beats1 = v > m1
                beats2 = v > m2
                ec = jnp.full((16,), e, jnp.int32)
                i2 = jnp.where(beats1, i1, jnp.where(beats2, ec, i2))
                m2 = jnp.where(beats1, m1, jnp.where(beats2, v, m2))
                i1 = jnp.where(beats1, ec, i1)
                m1 = jnp.where(beats1, v, m1)
            ex = jnp.exp(m2 - m1)
            s1 = 1.0 / (1.0 + ex)
            s2 = ex * s1
            i1_v[sl] = i1
            i2_v[sl] = i2
            s1_v[sl] = s1
            s2_v[sl] = s2

        pltpu.sync_copy(i1_v, i1_hbm.at[pl.ds(base, per_w)])
        pltpu.sync_copy(i2_v, i2_hbm.at[pl.ds(base, per_w)])
        pltpu.sync_copy(s1_v, s1_hbm.at[pl.ds(base, per_w)])
        pltpu.sync_copy(s2_v, s2_hbm.at[pl.ds(base, per_w)])

    return sc_topk


def kernel(inp, W, b):
    n_tok, _ = inp.shape
    n_expert = W.shape[0]
    info = plsc.get_sparse_core_info()
    nc, ns = info.num_cores, info.num_subcores
    gate = _gate_matmul(inp, W, b, nc * ns)
    i1, i2, s1, s2 = _make_sc_topk(n_tok, n_expert, nc, ns)(gate)
    return jnp.stack([i1, i2], axis=1), jnp.stack([s1, s2], axis=1)


# speedup vs baseline: 1.7231x; 1.7231x over previous
import functools

import jax
import jax.numpy as jnp
from jax.experimental import pallas as pl


def _router_kernel(x_ref, w_ref, b_ref, idx_ref, score_ref, *, n_expert):
    x = x_ref[...]
    w = w_ref[...]
    gate = jax.lax.dot_general(
        w, x,
        dimension_numbers=(((1,), (1,)), ((), ())),
        preferred_element_type=jnp.float32,
    )
    gate = gate + b_ref[:, :1]

    tile = gate.shape[1]
    sub = jax.lax.broadcasted_iota(jnp.int32, (n_expert, tile), 0)

    m1 = jnp.max(gate, axis=0, keepdims=True)
    i1 = jnp.min(jnp.where(gate == m1, sub, n_expert), axis=0, keepdims=True)
    masked = jnp.where(sub == i1, -jnp.inf, gate)
    m2 = jnp.max(masked, axis=0, keepdims=True)
    i2 = jnp.min(jnp.where(masked == m2, sub, n_expert), axis=0, keepdims=True)

    e = jnp.exp(m2 - m1)
    denom = 1.0 + e
    s1 = 1.0 / denom
    s2 = e / denom

    idx_ref[...] = jnp.concatenate([i1, i2], axis=0)
    score_ref[...] = jnp.concatenate([s1, s2], axis=0)


def kernel(inp, W, b):
    n_tok, d_model = inp.shape
    n_expert = W.shape[0]
    tile = 1024

    grid = (n_tok // tile,)
    out_idx, out_score = pl.pallas_call(
        functools.partial(_router_kernel, n_expert=n_expert),
        grid=grid,
        in_specs=[
            pl.BlockSpec((tile, d_model), lambda i: (i, 0)),
            pl.BlockSpec((n_expert, d_model), lambda i: (0, 0)),
            pl.BlockSpec((n_expert, 128), lambda i: (0, 0)),
        ],
        out_specs=[
            pl.BlockSpec((2, tile), lambda i: (0, i)),
            pl.BlockSpec((2, tile), lambda i: (0, i)),
        ],
        out_shape=[
            jax.ShapeDtypeStruct((2, n_tok), jnp.int32),
            jax.ShapeDtypeStruct((2, n_tok), jnp.float32),
        ],
    )(inp, W, jnp.broadcast_to(b[:, None], (n_expert, 128)))
    return out_idx.T, out_score.T
